# baseline (speedup 1.0000x reference)
import functools

import jax
import jax.numpy as jnp
from jax.experimental import pallas as pl
from jax.experimental.pallas import tpu as pltpu

F32 = jnp.float32
BF16 = jnp.bfloat16
LANE = 128


def _dot(a, b, precision=None):
    return jax.lax.dot_general(a, b, (((a.ndim - 1,), (0,)), ((), ())),
                               precision=precision, preferred_element_type=F32)


def _gelu(v):
    return 0.5 * v * (1.0 + jax.lax.erf(v * 0.7071067811865476))


def _pre_kernel(x_ref, wt_ref, bt_ref, xe_ref):
    xe = _dot(x_ref[...], wt_ref[...])
    xe_ref[...] = _gelu(xe + bt_ref[...])


def _gate_kernel(xe_ref, emb_ref, wg_ref, bgp_ref, xb_ref, wd_ref):
    bm = xe_ref.shape[0]
    lanes = jax.lax.broadcasted_iota(jnp.int32, (bm, LANE), 1)
    xf = xe_ref[...] + emb_ref[0]
    xb_ref[...] = xf.astype(BF16)
    logits = _dot(xf.astype(BF16), wg_ref[...]) + bgp_ref[...]
    m = jnp.max(logits, axis=1, keepdims=True)
    ex = jnp.exp(logits - m)
    probs = ex / jnp.sum(ex, axis=1, keepdims=True)
    v1 = jnp.max(probs, axis=1, keepdims=True)
    f1 = jnp.min(jnp.where(probs == v1, lanes, LANE), axis=1, keepdims=True)
    sel1 = lanes == f1
    p2 = jnp.where(sel1, -1.0, probs)
    v2 = jnp.max(p2, axis=1, keepdims=True)
    f2 = jnp.min(jnp.where(p2 == v2, lanes, LANE), axis=1, keepdims=True)
    sel2 = lanes == f2
    wd_ref[...] = jnp.where(sel1, v1, 0.0) + jnp.where(sel2, v2, 0.0)


def _moe_kernel(xb_ref, wd_ref, w1_ref, b1_ref, w2_ref, b2_ref, out_ref):
    e = pl.program_id(1)
    bm = xb_ref.shape[0]
    lanes = jax.lax.broadcasted_iota(jnp.int32, (bm, LANE), 1)
    w_col = jnp.sum(wd_ref[...] * jnp.where(lanes == e, 1.0, 0.0), axis=1,
                    keepdims=True)
    h = _gelu(_dot(xb_ref[...], w1_ref[0]) + b1_ref[0])
    wh = (h * w_col).astype(BF16)
    acc = _dot(wh, w2_ref[0]) + w_col * b2_ref[0]

    @pl.when(e == 0)
    def _init():
        out_ref[...] = acc

    @pl.when(e != 0)
    def _acc():
        out_ref[...] += acc


def _post_kernel(moe_ref, xe_ref, emb_ref, g_ref, be_ref, wp_ref, bp_ref,
                 out_ref):
    mo = moe_ref[...]
    m = jnp.mean(mo, axis=1, keepdims=True)
    v = jnp.mean((mo - m) ** 2, axis=1, keepdims=True)
    ln = (mo - m) / jnp.sqrt(v + 1e-5) * g_ref[0] + be_ref[0]
    y = _gelu(ln) + (xe_ref[...] + emb_ref[0])
    out_ref[...] = _dot(y.astype(BF16), wp_ref[0]) + bp_ref[0]


def kernel(x, W_txt, b_txt, l2_emb, cl_emb, Wg, bg, W1, b1, W2, b2,
           g_l2, be_l2, g_cl, be_cl, W_t2v, b_t2v, W_cl, b_cl):
    B, S, TD = x.shape
    H = W_txt.shape[1]
    E = Wg.shape[1]
    SD = W_t2v.shape[1]
    T = B * S
    bm = min(1024, T)
    nrb = T // bm
    nb = 2 * nrb

    xf = x.reshape(T, TD).astype(BF16)

    xe = pl.pallas_call(
        _pre_kernel,
        grid=(nrb,),
        in_specs=[
            pl.BlockSpec((bm, TD), lambda i: (i, 0)),
            pl.BlockSpec((TD, H), lambda i: (0, 0)),
            pl.BlockSpec((1, H), lambda i: (0, 0)),
        ],
        out_specs=pl.BlockSpec((bm, H), lambda i: (i, 0)),
        out_shape=jax.ShapeDtypeStruct((T, H), F32),
    )(xf, W_txt.astype(BF16), b_txt.reshape(1, H))

    emb = jnp.concatenate([l2_emb.reshape(1, 1, H), cl_emb.reshape(1, 1, H)],
                          axis=0)
    wg_pad = jnp.zeros((H, LANE), F32).at[:, :E].set(Wg).astype(BF16)
    bg_pad = jnp.full((1, LANE), -1e30, F32).at[0, :E].set(bg)
    w1b = W1.astype(BF16)
    w2b = W2.astype(BF16)

    xb, wd = pl.pallas_call(
        _gate_kernel,
        grid=(nb,),
        in_specs=[
            pl.BlockSpec((bm, H), lambda i: (i % nrb, 0)),
            pl.BlockSpec((1, 1, H), lambda i: (i // nrb, 0, 0)),
            pl.BlockSpec((H, LANE), lambda i: (0, 0)),
            pl.BlockSpec((1, LANE), lambda i: (0, 0)),
        ],
        out_specs=[
            pl.BlockSpec((bm, H), lambda i: (i, 0)),
            pl.BlockSpec((bm, LANE), lambda i: (i, 0)),
        ],
        out_shape=[
            jax.ShapeDtypeStruct((2 * T, H), BF16),
            jax.ShapeDtypeStruct((2 * T, LANE), F32),
        ],
    )(xe, emb, wg_pad, bg_pad)

    bmb = min(2048, 2 * T)
    nbb = (2 * T) // bmb
    moe = pl.pallas_call(
        _moe_kernel,
        grid=(nbb, E),
        in_specs=[
            pl.BlockSpec((bmb, H), lambda i, e: (i, 0)),
            pl.BlockSpec((bmb, LANE), lambda i, e: (i, 0)),
            pl.BlockSpec((1, H, H), lambda i, e: (e, 0, 0)),
            pl.BlockSpec((1, 1, H), lambda i, e: (e, 0, 0)),
            pl.BlockSpec((1, H, H), lambda i, e: (e, 0, 0)),
            pl.BlockSpec((1, 1, H), lambda i, e: (e, 0, 0)),
        ],
        out_specs=pl.BlockSpec((bmb, H), lambda i, e: (i, 0)),
        out_shape=jax.ShapeDtypeStruct((2 * T, H), F32),
        compiler_params=pltpu.CompilerParams(
            dimension_semantics=("arbitrary", "arbitrary")),
    )(xb, wd, w1b, b1.reshape(E, 1, H), w2b, b2.reshape(E, 1, H))

    g2 = jnp.concatenate([g_l2.reshape(1, 1, H), g_cl.reshape(1, 1, H)], 0)
    be2 = jnp.concatenate([be_l2.reshape(1, 1, H), be_cl.reshape(1, 1, H)], 0)
    wp = jnp.stack([W_t2v, W_cl], axis=0).astype(BF16)
    bp = jnp.concatenate([b_t2v.reshape(1, 1, SD), b_cl.reshape(1, 1, H)], 0)

    out = pl.pallas_call(
        _post_kernel,
        grid=(nb,),
        in_specs=[
            pl.BlockSpec((bm, H), lambda i: (i, 0)),
            pl.BlockSpec((bm, H), lambda i: (i % nrb, 0)),
            pl.BlockSpec((1, 1, H), lambda i: (i // nrb, 0, 0)),
            pl.BlockSpec((1, 1, H), lambda i: (i // nrb, 0, 0)),
            pl.BlockSpec((1, 1, H), lambda i: (i // nrb, 0, 0)),
            pl.BlockSpec((1, H, H), lambda i: (i // nrb, 0, 0)),
            pl.BlockSpec((1, 1, H), lambda i: (i // nrb, 0, 0)),
        ],
        out_specs=pl.BlockSpec((bm, H), lambda i: (i, 0)),
        out_shape=jax.ShapeDtypeStruct((2 * T, H), F32),
    )(moe, xe, emb, g2, be2, wp, bp)

    return (out[:T].reshape(B, S, SD), out[T:].reshape(B, S, H))


# speedup vs baseline: 1.0734x; 1.0734x over previous
import functools

import jax
import jax.numpy as jnp
from jax.experimental import pallas as pl
from jax.experimental.pallas import tpu as pltpu

F32 = jnp.float32
BF16 = jnp.bfloat16
LANE = 128


def _dot(a, b, precision=None):
    return jax.lax.dot_general(a, b, (((a.ndim - 1,), (0,)), ((), ())),
                               precision=precision, preferred_element_type=F32)


def _gelu(v):
    return 0.5 * v * (1.0 + jax.lax.erf(v * 0.7071067811865476))


def _pre_kernel(x_ref, wt_ref, bt_ref, xe_ref):
    xe = _dot(x_ref[...], wt_ref[...])
    xe_ref[...] = _gelu(xe + bt_ref[...])


def _moe_kernel(nrb, ne,
                xe_ref, emb_ref, wg_ref, bgp_ref, w1_ref, b1_ref, w2_ref,
                b2_ref, g_ref, be_ref, wp_ref, bp_ref, out_ref, xb_s, wd_s,
                acc_s):
    e = pl.program_id(1)
    bm = xe_ref.shape[0]
    lanes = jax.lax.broadcasted_iota(jnp.int32, (bm, LANE), 1)

    @pl.when(e == 0)
    def _gate():
        xf = xe_ref[...] + emb_ref[0]
        xb_s[...] = xf.astype(BF16)
        logits = _dot(xf.astype(BF16), wg_ref[...])
        logits = logits + bgp_ref[...]
        m = jnp.max(logits, axis=1, keepdims=True)
        ex = jnp.exp(logits - m)
        probs = ex / jnp.sum(ex, axis=1, keepdims=True)
        v1 = jnp.max(probs, axis=1, keepdims=True)
        f1 = jnp.min(jnp.where(probs == v1, lanes, LANE), axis=1, keepdims=True)
        sel1 = lanes == f1
        p2 = jnp.where(sel1, -1.0, probs)
        v2 = jnp.max(p2, axis=1, keepdims=True)
        f2 = jnp.min(jnp.where(p2 == v2, lanes, LANE), axis=1, keepdims=True)
        sel2 = lanes == f2
        wd_s[...] = jnp.where(sel1, v1, 0.0) + jnp.where(sel2, v2, 0.0)

    xb = xb_s[...]
    h = _gelu(_dot(xb, w1_ref[0]) + b1_ref[0])
    contrib = _dot(h.astype(BF16), w2_ref[0]) + b2_ref[0]
    w_col = jnp.sum(wd_s[...] * jnp.where(lanes == e, 1.0, 0.0), axis=1,
                    keepdims=True)
    acc = w_col * contrib

    @pl.when(e == 0)
    def _init():
        acc_s[...] = acc

    @pl.when((e != 0) & (e != ne - 1))
    def _acc():
        acc_s[...] += acc

    @pl.when(e == ne - 1)
    def _finish():
        mo = acc_s[...] + acc
        m = jnp.mean(mo, axis=1, keepdims=True)
        v = jnp.mean((mo - m) ** 2, axis=1, keepdims=True)
        ln = (mo - m) / jnp.sqrt(v + 1e-5) * g_ref[0] + be_ref[0]
        y = _gelu(ln) + (xe_ref[...] + emb_ref[0])
        out_ref[...] = _dot(y.astype(BF16), wp_ref[0]) + bp_ref[0]


def _post_kernel(moe_ref, xe_ref, emb_ref, g_ref, be_ref, wp_ref, bp_ref,
                 out_ref):
    mo = moe_ref[...]
    m = jnp.mean(mo, axis=1, keepdims=True)
    v = jnp.mean((mo - m) ** 2, axis=1, keepdims=True)
    ln = (mo - m) / jnp.sqrt(v + 1e-5) * g_ref[0] + be_ref[0]
    y = _gelu(ln) + (xe_ref[...] + emb_ref[0])
    out_ref[...] = _dot(y.astype(BF16), wp_ref[0]) + bp_ref[0]


def kernel(x, W_txt, b_txt, l2_emb, cl_emb, Wg, bg, W1, b1, W2, b2,
           g_l2, be_l2, g_cl, be_cl, W_t2v, b_t2v, W_cl, b_cl):
    B, S, TD = x.shape
    H = W_txt.shape[1]
    E = Wg.shape[1]
    SD = W_t2v.shape[1]
    T = B * S
    bm = min(1024, T)
    nrb = T // bm
    nb = 2 * nrb

    xf = x.reshape(T, TD).astype(BF16)

    xe = pl.pallas_call(
        _pre_kernel,
        grid=(nrb,),
        in_specs=[
            pl.BlockSpec((bm, TD), lambda i: (i, 0)),
            pl.BlockSpec((TD, H), lambda i: (0, 0)),
            pl.BlockSpec((1, H), lambda i: (0, 0)),
        ],
        out_specs=pl.BlockSpec((bm, H), lambda i: (i, 0)),
        out_shape=jax.ShapeDtypeStruct((T, H), F32),
    )(xf, W_txt.astype(BF16), b_txt.reshape(1, H))

    emb = jnp.concatenate([l2_emb.reshape(1, 1, H), cl_emb.reshape(1, 1, H)],
                          axis=0)
    wg_pad = jnp.zeros((H, LANE), F32).at[:, :E].set(Wg).astype(BF16)
    bg_pad = jnp.full((1, LANE), -1e30, F32).at[0, :E].set(bg)
    w1b = W1.astype(BF16)
    w2b = W2.astype(BF16)

    g2 = jnp.concatenate([g_l2.reshape(1, 1, H), g_cl.reshape(1, 1, H)], 0)
    be2 = jnp.concatenate([be_l2.reshape(1, 1, H), be_cl.reshape(1, 1, H)], 0)
    wp = jnp.stack([W_t2v, W_cl], axis=0).astype(BF16)
    bp = jnp.concatenate([b_t2v.reshape(1, 1, SD), b_cl.reshape(1, 1, H)], 0)

    out = pl.pallas_call(
        functools.partial(_moe_kernel, nrb, E),
        grid=(nb, E),
        in_specs=[
            pl.BlockSpec((bm, H), lambda i, e: (i % nrb, 0)),
            pl.BlockSpec((1, 1, H), lambda i, e: (i // nrb, 0, 0)),
            pl.BlockSpec((H, LANE), lambda i, e: (0, 0)),
            pl.BlockSpec((1, LANE), lambda i, e: (0, 0)),
            pl.BlockSpec((1, H, H), lambda i, e: (e, 0, 0)),
            pl.BlockSpec((1, 1, H), lambda i, e: (e, 0, 0)),
            pl.BlockSpec((1, H, H), lambda i, e: (e, 0, 0)),
            pl.BlockSpec((1, 1, H), lambda i, e: (e, 0, 0)),
            pl.BlockSpec((1, 1, H), lambda i, e: (i // nrb, 0, 0)),
            pl.BlockSpec((1, 1, H), lambda i, e: (i // nrb, 0, 0)),
            pl.BlockSpec((1, H, H), lambda i, e: (i // nrb, 0, 0)),
            pl.BlockSpec((1, 1, H), lambda i, e: (i // nrb, 0, 0)),
        ],
        out_specs=pl.BlockSpec((bm, H), lambda i, e: (i, 0)),
        out_shape=jax.ShapeDtypeStruct((2 * T, H), F32),
        scratch_shapes=[
            pltpu.VMEM((bm, H), BF16),
            pltpu.VMEM((bm, LANE), F32),
            pltpu.VMEM((bm, H), F32),
        ],
        compiler_params=pltpu.CompilerParams(
            dimension_semantics=("arbitrary", "arbitrary")),
    )(xe, emb, wg_pad, bg_pad, w1b, b1.reshape(E, 1, H), w2b,
      b2.reshape(E, 1, H), g2, be2, wp, bp)

    return (out[:T].reshape(B, S, SD), out[T:].reshape(B, S, H))
